# baseline (speedup 1.0000x reference)
import functools

import jax
import jax.numpy as jnp
from jax import lax
from jax.experimental import pallas as pl
from jax.experimental.pallas import tpu as pltpu
from jax.experimental.pallas import tpu_sc as plsc

NUM_ENTITIES = 1_000_000
DIM = 64
NI, NJ = 16384, 26
B = NI * NJ
NC, NS = 2, 16
NW = NC * NS

CW = 384
NPANEL = NUM_ENTITIES // CW
TAIL = NUM_ENTITIES - NPANEL * CW
PPW = -(-NPANEL // NW)

K = 256
IB = NI // K
M = (NJ * IB) // NW
BPW = M * K

_mesh = plsc.VectorSubcoreMesh(core_axis_name="c", subcore_axis_name="s")


def _wid():
    return lax.axis_index("s") * NC + lax.axis_index("c")


@functools.partial(
    pl.kernel,
    mesh=_mesh,
    out_type=jax.ShapeDtypeStruct((NUM_ENTITIES // 2, 2 * DIM), jnp.float32),
    compiler_params=pltpu.CompilerParams(needs_layout_passes=False),
    scratch_types=[
        pltpu.VMEM((2, DIM, CW), jnp.float32),
        pltpu.VMEM((2, CW // 2, 2 * DIM), jnp.float32),
        pltpu.VMEM((TAIL // 2, 2 * DIM), jnp.float32),
        pltpu.SemaphoreType.DMA,
        pltpu.SemaphoreType.DMA,
        pltpu.SemaphoreType.DMA,
        pltpu.SemaphoreType.DMA,
    ],
)
def _repack32(wt_hbm, wtail_hbm, w128_hbm, in_v, out_v, tail_v,
              isem0, isem1, osem0, osem1):
    wid = _wid()
    isems = (isem0, isem1)
    osems = (osem0, osem1)

    def _pid(k):
        return wid + k * NW

    def _fire_in(k, s):
        c0 = _pid(k) * CW
        pltpu.async_copy(wt_hbm.at[:, pl.ds(c0, CW)], in_v.at[s], isems[s])

    def _wait_in(s):
        pltpu.make_async_copy(wt_hbm.at[:, pl.ds(0, CW)],
                              in_v.at[s], isems[s]).wait()

    def _fire_out(k, s):
        r0 = _pid(k) * (CW // 2)
        pltpu.async_copy(out_v.at[s], w128_hbm.at[pl.ds(r0, CW // 2)],
                         osems[s])

    def _wait_out(s):
        pltpu.make_async_copy(out_v.at[s], w128_hbm.at[pl.ds(0, CW // 2)],
                              osems[s]).wait()

    @pl.when(wid == NW - 1)
    def _():
        pltpu.sync_copy(wtail_hbm, tail_v)
        pltpu.sync_copy(tail_v, w128_hbm.at[pl.ds(NPANEL * (CW // 2),
                                                  TAIL // 2)])

    @pl.when(_pid(0) < NPANEL)
    def _():
        _fire_in(0, 0)

    @pl.when(_pid(1) < NPANEL)
    def _():
        _fire_in(1, 1)

    def _panel(k, _):
        for s in (0, 1):
            kk = 2 * k + s

            @pl.when(_pid(kk) < NPANEL)
            def _():
                _wait_in(s)

                @pl.when(kk >= 2)
                def _():
                    _wait_out(s)

                dvecs = [lax.iota(jnp.int32, 16) + (t % 4) * 16
                         for t in range(8)]

                def _q(q, _):
                    c = 2 * q
                    vals = [
                        plsc.load_gather(
                            in_v.at[s],
                            [dvecs[t],
                             jnp.full((16,), t // 4, jnp.int32) + c])
                        for t in range(8)
                    ]
                    for t in range(8):
                        out_v[s, q, pl.ds(t * 16, 16)] = vals[t]
                    return _
                lax.fori_loop(0, CW // 2, _q, None)

                _fire_out(kk, s)

                @pl.when(_pid(kk + 2) < NPANEL)
                def _():
                    _fire_in(kk + 2, s)
        return _
    lax.fori_loop(0, PPW // 2, _panel, None)

    @pl.when(_pid(PPW - 2) < NPANEL)
    def _():
        _wait_out((PPW - 2) % 2)

    @pl.when(_pid(PPW - 1) < NPANEL)
    def _():
        _wait_out((PPW - 1) % 2)


@functools.partial(
    pl.kernel,
    mesh=_mesh,
    out_type=jax.ShapeDtypeStruct((NJ, DIM, NI), jnp.float32),
    compiler_params=pltpu.CompilerParams(needs_layout_passes=False),
    scratch_types=[
        pltpu.VMEM((BPW,), jnp.int32),
        pltpu.VMEM((BPW,), jnp.int32),
        pltpu.VMEM((2, K, 2 * DIM), jnp.float32),
        pltpu.VMEM((2, DIM, K), jnp.float32),
        pltpu.SemaphoreType.DMA,
        pltpu.SemaphoreType.DMA,
        pltpu.SemaphoreType.DMA,
        pltpu.SemaphoreType.DMA,
    ],
)
def _embed32(ids_hbm, w128_hbm, out_hbm, idx_v, ihi_v, g_v, t_v,
             gsem0, gsem1, ssem0, ssem1):
    wid = _wid()
    base = wid * BPW

    pltpu.sync_copy(ids_hbm.at[pl.ds(base, BPW)], idx_v)

    def _pre(k, _):
        sl = pl.ds(k * 16, 16)
        ihi_v[sl] = idx_v[sl] >> 1
        return _
    lax.fori_loop(0, BPW // 16, _pre, None)

    gsems = (gsem0, gsem1)
    ssems = (ssem0, ssem1)

    def _fire_gather(m, s):
        pltpu.async_copy(
            w128_hbm.at[ihi_v.at[pl.ds(m * K, K)]], g_v.at[s], gsems[s])

    def _wait_gather(s):
        pltpu.make_async_copy(w128_hbm.at[ihi_v.at[pl.ds(0, K)]],
                              g_v.at[s], gsems[s]).wait()

    def _out_slice(m):
        gm = wid * M + m
        j = gm // IB
        i0 = (gm % IB) * K
        return out_hbm.at[j, :, pl.ds(i0, K)]

    def _fire_store(m, s):
        pltpu.async_copy(t_v.at[s], _out_slice(m), ssems[s])

    def _wait_store(s):
        pltpu.make_async_copy(t_v.at[s], _out_slice(0), ssems[s]).wait()

    _fire_gather(0, 0)
    _fire_gather(1, 1)

    def _item(i, _):
        for s in (0, 1):
            m = 2 * i + s
            _wait_gather(s)

            @pl.when(m >= 2)
            def _():
                _wait_store(s)

            def _grp(g, _):
                sl = pl.ds(m * K + g * 16, 16)
                h64 = (idx_v[sl] & 1) << 6
                rvec = lax.iota(jnp.int32, 16) + g * 16
                for d0 in range(0, DIM, 16):
                    vals = [
                        plsc.load_gather(g_v.at[s], [rvec, h64 + (d0 + t)])
                        for t in range(16)
                    ]
                    for t in range(16):
                        t_v[s, d0 + t, pl.ds(g * 16, 16)] = vals[t]
                return _
            lax.fori_loop(0, K // 16, _grp, None)

            _fire_store(m, s)

            @pl.when(m + 2 < M)
            def _():
                _fire_gather(m + 2, s)
        return _
    lax.fori_loop(0, M // 2, _item, None)

    _wait_store(0)
    _wait_store(1)


def kernel(ids, weight):
    ids_lin = jnp.transpose(ids).reshape(-1)
    wt = jnp.transpose(weight)
    nt = NPANEL * CW
    wtail = weight[nt:].reshape(TAIL // 2, 2 * DIM)
    w128 = _repack32(wt, wtail)
    out_t = _embed32(ids_lin, w128)
    return jnp.transpose(out_t, (2, 0, 1))


# speedup vs baseline: 1.4785x; 1.4785x over previous
import functools

import jax
import jax.numpy as jnp
from jax import lax
from jax.experimental import pallas as pl
from jax.experimental.pallas import tpu as pltpu
from jax.experimental.pallas import tpu_sc as plsc

NUM_ENTITIES = 1_000_000
DIM = 64
NI, NJ = 16384, 26
B = NI * NJ
NC, NS = 2, 16
NW = NC * NS
K = 256
IB = NI // K
M = (NJ * IB) // NW
BPW = M * K

_mesh = plsc.VectorSubcoreMesh(core_axis_name="c", subcore_axis_name="s")


@functools.partial(
    pl.kernel,
    mesh=_mesh,
    out_type=jax.ShapeDtypeStruct((NJ, DIM, NI), jnp.float32),
    compiler_params=pltpu.CompilerParams(needs_layout_passes=False),
    scratch_types=[
        pltpu.VMEM((BPW,), jnp.int32),
        pltpu.VMEM((BPW,), jnp.int32),
        pltpu.VMEM((2, K, 2 * DIM), jnp.float32),
        pltpu.VMEM((2, DIM, K), jnp.float32),
        pltpu.SemaphoreType.DMA,
        pltpu.SemaphoreType.DMA,
        pltpu.SemaphoreType.DMA,
        pltpu.SemaphoreType.DMA,
    ],
)
def _embed32(ids_hbm, w128_hbm, out_hbm, idx_v, ihi_v, g_v, t_v,
             gsem0, gsem1, ssem0, ssem1):
    wid = lax.axis_index("s") * NC + lax.axis_index("c")
    base = wid * BPW

    pltpu.sync_copy(ids_hbm.at[pl.ds(base, BPW)], idx_v)

    def _pre(k, _):
        sl = pl.ds(k * 16, 16)
        ihi_v[sl] = idx_v[sl] >> 1
        return _
    lax.fori_loop(0, BPW // 16, _pre, None)

    gsems = (gsem0, gsem1)
    ssems = (ssem0, ssem1)

    def _fire_gather(m, s):
        pltpu.async_copy(
            w128_hbm.at[ihi_v.at[pl.ds(m * K, K)]], g_v.at[s], gsems[s])

    def _wait_gather(s):
        pltpu.make_async_copy(w128_hbm.at[ihi_v.at[pl.ds(0, K)]],
                              g_v.at[s], gsems[s]).wait()

    def _out_slice(m):
        gm = wid * M + m
        j = gm // IB
        i0 = (gm % IB) * K
        return out_hbm.at[j, :, pl.ds(i0, K)]

    def _fire_store(m, s):
        pltpu.async_copy(t_v.at[s], _out_slice(m), ssems[s])

    def _wait_store(s):
        pltpu.make_async_copy(t_v.at[s], _out_slice(0), ssems[s]).wait()

    _fire_gather(0, 0)
    _fire_gather(1, 1)

    iota16 = lax.iota(jnp.int32, 16)

    def _item(i, _):
        for s in (0, 1):
            m = 2 * i + s
            _wait_gather(s)

            @pl.when(m >= 2)
            def _():
                _wait_store(s)

            def _grp(g2, _):
                g = 2 * g2
                sl0 = pl.ds(m * K + g * 16, 16)
                sl1 = pl.ds(m * K + g * 16 + 16, 16)
                h0 = (idx_v[sl0] & 1) << 6
                h1 = (idx_v[sl1] & 1) << 6
                r0 = iota16 + g * 16
                r1 = iota16 + (g * 16 + 16)
                for d0 in range(0, DIM, 16):
                    vals = (
                        [plsc.load_gather(g_v.at[s], [r0, h0 + (d0 + t)])
                         for t in range(16)]
                        + [plsc.load_gather(g_v.at[s], [r1, h1 + (d0 + t)])
                           for t in range(16)]
                    )
                    for t in range(16):
                        t_v[s, d0 + t, pl.ds(g * 16, 16)] = vals[t]
                    for t in range(16):
                        t_v[s, d0 + t, pl.ds(g * 16 + 16, 16)] = vals[16 + t]
                return _
            lax.fori_loop(0, K // 32, _grp, None)

            _fire_store(m, s)

            @pl.when(m + 2 < M)
            def _():
                _fire_gather(m + 2, s)
        return _
    lax.fori_loop(0, M // 2, _item, None)

    _wait_store(0)
    _wait_store(1)


def kernel(ids, weight):
    ids_lin = jnp.transpose(ids).reshape(-1)
    w128 = weight.reshape(NUM_ENTITIES // 2, 2 * DIM)
    out_t = _embed32(ids_lin, w128)
    return jnp.transpose(out_t, (2, 0, 1))
